# baseline (speedup 1.0000x reference)
import functools

import numpy as np
import jax
import jax.numpy as jnp
from jax import lax
from jax.experimental import pallas as pl
from jax.experimental.pallas import tpu as pltpu
from jax.experimental.pallas import tpu_sc as plsc

_NF = 26
_ROWS_PER_FIELD = 100000
_BATCH = 16384
_B = _BATCH * _NF
_D = 32
_PACK = 4
_VP = 650000
_NW = 32
_BPW = _B // _NW
_BATCH_PW = _BATCH // _NW
_C = 128
_NCHUNK = _BPW // _C
_VL = 16
_P = 208
_NJ = _P // _VL
_NBLK = _BPW // _P

_k = np.arange(_P, dtype=np.int32)
_CONSTS = np.concatenate([
    ((_k % _NF) * _BATCH_PW + _k // _NF).reshape(_NJ, _VL).ravel(),
    (((_k % _NF) * _ROWS_PER_FIELD) >> 2).reshape(_NJ, _VL).ravel(),
])

_mesh = plsc.VectorSubcoreMesh(core_axis_name="c", subcore_axis_name="s")


@functools.partial(
    pl.kernel,
    out_type=jax.ShapeDtypeStruct((_B, _D), jnp.float32),
    mesh=_mesh,
    compiler_params=pltpu.CompilerParams(
        use_tc_tiling_on_sc=True, needs_layout_passes=False
    ),
    scratch_types=[
        pltpu.VMEM((_BPW,), jnp.int32),
        pltpu.VMEM((2 * _P,), jnp.int32),
        pltpu.VMEM((_BPW,), jnp.int32),
        pltpu.VMEM((_BPW,), jnp.int32),
        pltpu.VMEM((_C, _PACK * _D), jnp.float32),
        pltpu.VMEM((_C, _PACK * _D), jnp.float32),
        pltpu.VMEM((_C, _D), jnp.float32),
        pltpu.VMEM((_C, _D), jnp.float32),
        pltpu.SemaphoreType.DMA,
        pltpu.SemaphoreType.DMA,
        pltpu.SemaphoreType.DMA,
        pltpu.SemaphoreType.DMA,
    ],
)
def _embed_gather(tpk_hbm, xt_hbm, consts_hbm, out_hbm,
                  stage_v, consts_v, pidx_v, off_v,
                  pk0, pk1, ex0, ex1,
                  gsem0, gsem1, osem0, osem1):
    wid = lax.axis_index("s") * 2 + lax.axis_index("c")
    base = wid * _BPW

    pltpu.sync_copy(consts_hbm, consts_v)
    for f in range(_NF):
        pltpu.sync_copy(
            xt_hbm.at[f].at[pl.ds(wid * _BATCH_PW, _BATCH_PW)],
            stage_v.at[pl.ds(f * _BATCH_PW, _BATCH_PW)],
        )

    for j in range(_NJ):
        avec = consts_v[pl.ds(j * _VL, _VL)]
        ovec = consts_v[pl.ds(_P + j * _VL, _VL)]

        def _blk(blk, carry, avec=avec, ovec=ovec, j=j):
            vals = plsc.load_gather(
                stage_v, [avec + jnp.full((_VL,), 8, jnp.int32) * blk]
            )
            s = pl.ds(blk * _P + j * _VL, _VL)
            pidx_v[s] = lax.shift_right_logical(vals, 2) + ovec
            off_v[s] = lax.shift_left(vals & 3, 5)
            return carry

        lax.fori_loop(0, _NBLK, _blk, 0)

    pks = (pk0, pk1)
    exs = (ex0, ex1)
    gsems = (gsem0, gsem1)
    osems = (osem0, osem1)
    iota = lax.broadcasted_iota(jnp.int32, (_VL,), 0)

    def _gather(g, par):
        return pltpu.async_copy(
            tpk_hbm.at[pidx_v.at[pl.ds(g * _C, _C)]], pks[par], gsems[par]
        )

    def _extract(g, par):
        gb = g * _C

        def _rows(r16, carry, par=par):
            rvec = iota + r16 * _VL
            offv = off_v[pl.ds(gb + r16 * _VL, _VL)]
            for j in range(_D):
                vals = plsc.load_gather(
                    pks[par], [rvec, offv + jnp.full((_VL,), j, jnp.int32)]
                )
                plsc.store_scatter(
                    exs[par], [rvec, jnp.full((_VL,), j, jnp.int32)], vals
                )
            return carry

        lax.fori_loop(0, _C // _VL, _rows, 0)

    def _wout(g, par):
        return pltpu.async_copy(
            exs[par], out_hbm.at[pl.ds(base + g * _C, _C)], osems[par]
        )

    _gather(0, 0).wait()
    ocp = [None, None]

    def _step(k, carry):
        g = k * 2

        @pl.when(g + 1 < _NCHUNK)
        def _():
            _gather(g + 1, 1)

        @pl.when(k > 0)
        def _():
            pltpu.make_async_copy(
                exs[0], out_hbm.at[pl.ds(base, _C)], osems[0]
            ).wait()

        _extract(g, 0)
        _wout(g, 0)

        @pl.when(g + 1 < _NCHUNK)
        def _():
            pltpu.make_async_copy(
                tpk_hbm.at[pidx_v.at[pl.ds(0, _C)]], pks[1], gsems[1]
            ).wait()

            @pl.when(g + 2 < _NCHUNK)
            def _():
                _gather(g + 2, 0)

            @pl.when(k > 0)
            def _():
                pltpu.make_async_copy(
                    exs[1], out_hbm.at[pl.ds(base, _C)], osems[1]
                ).wait()

            _extract(g + 1, 1)
            _wout(g + 1, 1)

            @pl.when(g + 2 < _NCHUNK)
            def _():
                pltpu.make_async_copy(
                    tpk_hbm.at[pidx_v.at[pl.ds(0, _C)]], pks[0], gsems[0]
                ).wait()

        return carry

    lax.fori_loop(0, _NCHUNK // 2, _step, 0)
    pltpu.make_async_copy(exs[0], out_hbm.at[pl.ds(base, _C)], osems[0]).wait()
    pltpu.make_async_copy(exs[1], out_hbm.at[pl.ds(base, _C)], osems[1]).wait()


def kernel(x, table):
    consts = jnp.asarray(_CONSTS)
    tpk = table.reshape(_VP, _PACK * _D)
    out = _embed_gather(tpk, x.T, consts)
    return out.reshape(_BATCH, _NF, _D)


# speedup vs baseline: 1.6526x; 1.6526x over previous
import functools

import numpy as np
import jax
import jax.numpy as jnp
from jax import lax
from jax.experimental import pallas as pl
from jax.experimental.pallas import tpu as pltpu
from jax.experimental.pallas import tpu_sc as plsc

_NF = 26
_ROWS_PER_FIELD = 100000
_BATCH = 16384
_B = _BATCH * _NF
_D = 32
_NW = 32
_BPW = _B // _NW
_BATCH_PW = _BATCH // _NW
_C = 208
_NCHUNK = _BPW // _C
_VL = 16
_P = 208
_NJ = _P // _VL
_NBLK = _BPW // _P

_k = np.arange(_P, dtype=np.int32)
_CONSTS = np.concatenate([
    ((_k % _NF) * _BATCH_PW + _k // _NF).reshape(_NJ, _VL).ravel(),
    ((_k % _NF) * _ROWS_PER_FIELD).reshape(_NJ, _VL).ravel(),
])

_mesh = plsc.VectorSubcoreMesh(core_axis_name="c", subcore_axis_name="s")


@functools.partial(
    pl.kernel,
    out_type=jax.ShapeDtypeStruct((_B, _D), jnp.float32),
    mesh=_mesh,
    compiler_params=pltpu.CompilerParams(
        use_tc_tiling_on_sc=True, needs_layout_passes=False
    ),
    scratch_types=[
        pltpu.VMEM((_BPW,), jnp.int32),
        pltpu.VMEM((2 * _P,), jnp.int32),
        pltpu.VMEM((_BPW,), jnp.int32),
        pltpu.VMEM((_C, _D), jnp.float32),
        pltpu.VMEM((_C, _D), jnp.float32),
        pltpu.SemaphoreType.DMA,
        pltpu.SemaphoreType.DMA,
        pltpu.SemaphoreType.DMA,
        pltpu.SemaphoreType.DMA,
    ],
)
def _embed_gather(table_hbm, xt_hbm, consts_hbm, out_hbm,
                  stage_v, consts_v, idx_v, rows0, rows1,
                  gsem0, gsem1, osem0, osem1):
    wid = lax.axis_index("s") * 2 + lax.axis_index("c")
    base = wid * _BPW

    pltpu.sync_copy(consts_hbm, consts_v)
    for f in range(_NF):
        pltpu.sync_copy(
            xt_hbm.at[f].at[pl.ds(wid * _BATCH_PW, _BATCH_PW)],
            stage_v.at[pl.ds(f * _BATCH_PW, _BATCH_PW)],
        )

    for j in range(_NJ):
        avec = consts_v[pl.ds(j * _VL, _VL)]
        ovec = consts_v[pl.ds(_P + j * _VL, _VL)]

        def _blk(blk, carry, avec=avec, ovec=ovec):
            vals = plsc.load_gather(
                stage_v, [avec + jnp.full((_VL,), 8, jnp.int32) * blk]
            )
            idx_v[pl.ds(blk * _P + j * _VL, _VL)] = vals + ovec
            return carry

        lax.fori_loop(0, _NBLK, _blk, 0)

    bufs = (rows0, rows1)
    gsems = (gsem0, gsem1)
    osems = (osem0, osem1)

    def _issue_chunk(g, par):
        def _vec(v, carry, par=par):
            vec = idx_v[pl.ds(g * _C + v * _VL, _VL)]
            for l in range(_VL):
                pltpu.async_copy(
                    table_hbm.at[vec[l]], bufs[par].at[v * _VL + l],
                    gsems[par],
                )
            return carry

        lax.fori_loop(0, _C // _VL, _vec, 0)

    def _drain_chunk(par):
        def _row(r, carry, par=par):
            pltpu.make_async_copy(
                table_hbm.at[0], bufs[par].at[0], gsems[par]
            ).wait()
            return carry

        lax.fori_loop(0, _C, _row, 0)

    def _wout(g, par):
        return pltpu.async_copy(
            bufs[par], out_hbm.at[pl.ds(base + g * _C, _C)], osems[par]
        )

    _issue_chunk(0, 0)
    for g in range(_NCHUNK):
        par = g % 2
        if g + 1 < _NCHUNK:
            if g >= 1:
                pltpu.make_async_copy(
                    bufs[1 - par], out_hbm.at[pl.ds(base, _C)], osems[1 - par]
                ).wait()
            _issue_chunk(g + 1, 1 - par)
        _drain_chunk(par)
        _wout(g, par)
    pltpu.make_async_copy(
        bufs[0], out_hbm.at[pl.ds(base, _C)], osems[0]
    ).wait()
    pltpu.make_async_copy(
        bufs[1], out_hbm.at[pl.ds(base, _C)], osems[1]
    ).wait()


def kernel(x, table):
    consts = jnp.asarray(_CONSTS)
    out = _embed_gather(table, x.T, consts)
    return out.reshape(_BATCH, _NF, _D)


# speedup vs baseline: 1.7009x; 1.0293x over previous
import functools

import numpy as np
import jax
import jax.numpy as jnp
from jax import lax
from jax.experimental import pallas as pl
from jax.experimental.pallas import tpu as pltpu
from jax.experimental.pallas import tpu_sc as plsc

_NF = 26
_ROWS_PER_FIELD = 100000
_BATCH = 16384
_B = _BATCH * _NF
_D = 32
_NW = 32
_BPW = _B // _NW
_BATCH_PW = _BATCH // _NW
_C = 208
_NCHUNK = _BPW // _C
_VL = 16
_P = 208
_NJ = _P // _VL
_NBLK = _BPW // _P

_k = np.arange(_P, dtype=np.int32)
_CONSTS = np.concatenate([
    ((_k % _NF) * _BATCH_PW + _k // _NF).reshape(_NJ, _VL).ravel(),
    ((_k % _NF) * _ROWS_PER_FIELD).reshape(_NJ, _VL).ravel(),
])

_mesh = plsc.VectorSubcoreMesh(core_axis_name="c", subcore_axis_name="s")


@functools.partial(
    pl.kernel,
    out_type=jax.ShapeDtypeStruct((_B, _D), jnp.float32),
    mesh=_mesh,
    compiler_params=pltpu.CompilerParams(
        use_tc_tiling_on_sc=True, needs_layout_passes=False
    ),
    scratch_types=[
        pltpu.VMEM((_BPW,), jnp.int32),
        pltpu.VMEM((2 * _P,), jnp.int32),
        pltpu.VMEM((_BPW,), jnp.int32),
        pltpu.VMEM((_C, _D), jnp.float32),
        pltpu.VMEM((_C, _D), jnp.float32),
        pltpu.SemaphoreType.DMA,
        pltpu.SemaphoreType.DMA,
        pltpu.SemaphoreType.DMA,
        pltpu.SemaphoreType.DMA,
    ],
)
def _embed_gather(table_hbm, xt_hbm, consts_hbm, out_hbm,
                  stage_v, consts_v, idx_v, rows0, rows1,
                  gsem0, gsem1, osem0, osem1):
    wid = lax.axis_index("s") * 2 + lax.axis_index("c")
    base = wid * _BPW

    pltpu.sync_copy(consts_hbm, consts_v)
    for f in range(_NF):
        pltpu.sync_copy(
            xt_hbm.at[f].at[pl.ds(wid * _BATCH_PW, _BATCH_PW)],
            stage_v.at[pl.ds(f * _BATCH_PW, _BATCH_PW)],
        )

    for j in range(_NJ):
        avec = consts_v[pl.ds(j * _VL, _VL)]
        ovec = consts_v[pl.ds(_P + j * _VL, _VL)]

        def _blk(blk, carry, avec=avec, ovec=ovec):
            vals = plsc.load_gather(
                stage_v, [avec + jnp.full((_VL,), 8, jnp.int32) * blk]
            )
            idx_v[pl.ds(blk * _P + j * _VL, _VL)] = vals + ovec
            return carry

        lax.fori_loop(0, _NBLK, _blk, 0)

    bufs = (rows0, rows1)
    gsems = (gsem0, gsem1)
    osems = (osem0, osem1)

    def _issue_chunk(g, par):
        def _vec(v, carry, par=par):
            vec = idx_v[pl.ds(g * _C + v * _VL, _VL)]
            for l in range(_VL):
                pltpu.async_copy(
                    table_hbm.at[vec[l]], bufs[par].at[v * _VL + l],
                    gsems[par],
                )
            return carry

        lax.fori_loop(0, _C // _VL, _vec, 0)

    def _drain_chunk(par):
        pltpu.make_async_copy(
            table_hbm.at[pl.ds(0, _C)], bufs[par], gsems[par]
        ).wait()

    def _wout(g, par):
        return pltpu.async_copy(
            bufs[par], out_hbm.at[pl.ds(base + g * _C, _C)], osems[par]
        )

    _issue_chunk(0, 0)
    for g in range(_NCHUNK):
        par = g % 2
        if g + 1 < _NCHUNK:
            if g >= 1:
                pltpu.make_async_copy(
                    bufs[1 - par], out_hbm.at[pl.ds(base, _C)], osems[1 - par]
                ).wait()
            _issue_chunk(g + 1, 1 - par)
        _drain_chunk(par)
        _wout(g, par)
    pltpu.make_async_copy(
        bufs[0], out_hbm.at[pl.ds(base, _C)], osems[0]
    ).wait()
    pltpu.make_async_copy(
        bufs[1], out_hbm.at[pl.ds(base, _C)], osems[1]
    ).wait()


def kernel(x, table):
    consts = jnp.asarray(_CONSTS)
    out = _embed_gather(table, x.T, consts)
    return out.reshape(_BATCH, _NF, _D)
